# baseline (speedup 1.0000x reference)
import functools

import jax
import jax.numpy as jnp
from jax import lax
from jax.experimental import pallas as pl
from jax.experimental.pallas import tpu as pltpu
from jax.experimental.pallas import tpu_sc as plsc


def _transform_body(x_ref, w_ref, b_ref, o_ref):
    x = x_ref[...]
    nrm = jnp.sqrt(jnp.sum(x * x, axis=1, keepdims=True))
    y = jnp.minimum(nrm, 1.0 - 1e-7)
    artanh = 0.5 * jnp.log((1.0 + y) / (1.0 - y))
    t = x * (artanh / jnp.maximum(nrm, 1e-15))
    o_ref[...] = (
        lax.dot_general(t, w_ref[...], (((1,), (1,)), ((), ())),
                        preferred_element_type=jnp.float32)
        + b_ref[...]
    )


def _transform(x, w, b, block_n):
    n, d_in = x.shape
    d_out = w.shape[0]
    return pl.pallas_call(
        _transform_body,
        grid=(n // block_n,),
        in_specs=[
            pl.BlockSpec((block_n, d_in), lambda i: (i, 0)),
            pl.BlockSpec((d_out, d_in), lambda i: (0, 0)),
            pl.BlockSpec((1, d_out), lambda i: (0, 0)),
        ],
        out_specs=pl.BlockSpec((block_n, d_out), lambda i: (i, 0)),
        out_shape=jax.ShapeDtypeStruct((n, d_out), jnp.float32),
    )(x, w, b.reshape(1, d_out))


def _make_seg_sum(n, n_pad, d, nw, n_chunks, chunk):
    mesh = plsc.VectorSubcoreMesh(core_axis_name="c", subcore_axis_name="s")
    ns = 16
    npt = (n_pad // ns) // 8 * 8
    remz = n_pad - ns * npt
    remo = n - ns * npt

    @functools.partial(
        pl.kernel,
        mesh=mesh,
        out_type=(
            jax.ShapeDtypeStruct((2, n, d), jnp.float32),
            jax.ShapeDtypeStruct((2, n_pad), jnp.float32),
        ),
        scratch_types=[
            pltpu.VMEM_SHARED((n_pad, d), jnp.float32),
            pltpu.VMEM_SHARED((n_pad,), jnp.float32),
            pltpu.VMEM((n_chunks, chunk), jnp.int32),
            pltpu.VMEM((n_chunks, chunk), jnp.int32),
            pltpu.VMEM((chunk, d), jnp.float32),
            pltpu.VMEM((128,), jnp.float32),
            pltpu.SemaphoreType.DMA,
            pltpu.SemaphoreType.DMA,
        ],
    )
    def seg_sum(t_hbm, src_hbm, dst_hbm, z2d_hbm, z1d_hbm,
                agg_hbm, cnt_hbm,
                acc_sh, cnt_sh, sidx_v, didx_v, rows_v, ones_v, gsem, csem):
        cc = lax.axis_index("c")
        s = lax.axis_index("s")
        wid = cc * ns + s

        pltpu.sync_copy(src_hbm.at[wid], sidx_v)
        pltpu.sync_copy(dst_hbm.at[wid], didx_v)

        pltpu.sync_copy(z2d_hbm.at[pl.ds(s * npt, npt)],
                        acc_sh.at[pl.ds(s * npt, npt)])

        @pl.when(s == 0)
        def _():
            pltpu.sync_copy(z1d_hbm, cnt_sh)
            if remz:
                pltpu.sync_copy(z2d_hbm.at[pl.ds(ns * npt, remz)],
                                acc_sh.at[pl.ds(ns * npt, remz)])

        for i in range(8):
            ones_v[pl.ds(i * 16, 16)] = jnp.full((16,), 1.0, jnp.float32)

        plsc.subcore_barrier()

        def body(j, carry):
            pltpu.async_copy(t_hbm.at[sidx_v.at[j]], rows_v, gsem).wait()
            pltpu.sync_copy(rows_v, acc_sh.at[didx_v.at[j]], add=True)
            pltpu.async_copy(ones_v.at[pl.ds(0, chunk)],
                             cnt_sh.at[didx_v.at[j]], csem, add=True)
            return carry

        lax.fori_loop(0, n_chunks, body, 0)

        def drain(j, carry):
            pltpu.make_async_copy(ones_v.at[pl.ds(0, chunk)],
                                  cnt_sh.at[didx_v.at[0]], csem).wait()
            return carry

        lax.fori_loop(0, n_chunks, drain, 0)

        plsc.subcore_barrier()

        pltpu.sync_copy(acc_sh.at[pl.ds(s * npt, npt)],
                        agg_hbm.at[cc, pl.ds(s * npt, npt)])

        @pl.when(s == 0)
        def _():
            pltpu.sync_copy(cnt_sh, cnt_hbm.at[cc])
            if remo:
                pltpu.sync_copy(acc_sh.at[pl.ds(ns * npt, remo)],
                                agg_hbm.at[cc, pl.ds(ns * npt, remo)])

    return seg_sum


def _finalize_body(agg_ref, cnt_ref, o_ref):
    a = agg_ref[0] + agg_ref[1]
    c = cnt_ref[...]
    csum = c[:, 0:1] + c[:, 1:2]
    neigh = a / jnp.maximum(csum, 1.0)
    nrm = jnp.sqrt(jnp.sum(neigh * neigh, axis=1, keepdims=True))
    o_ref[...] = jnp.tanh(nrm) * neigh / jnp.maximum(nrm, 1e-15)


def _finalize(agg, cnt_t, block_n):
    _, n, d = agg.shape
    return pl.pallas_call(
        _finalize_body,
        grid=(n // block_n,),
        in_specs=[
            pl.BlockSpec((2, block_n, d), lambda i: (0, i, 0)),
            pl.BlockSpec((block_n, 2), lambda i: (i, 0)),
        ],
        out_specs=pl.BlockSpec((block_n, d), lambda i: (i, 0)),
        out_shape=jax.ShapeDtypeStruct((n, d), jnp.float32),
    )(agg, cnt_t)


def kernel(x, edge_index, W, b):
    n, d_in = x.shape
    d_out = W.shape[0]
    e = edge_index.shape[1]

    nw = 32
    chunk = 128
    n_pad = n + 8
    epw = e // nw
    assert epw * nw == e
    epw_pad = -(-epw // (chunk * 4)) * (chunk * 4)
    n_chunks = epw_pad // chunk
    pad = epw_pad - epw

    transformed = _transform(x, W, b, block_n=1000)

    src = edge_index[0].reshape(nw, epw)
    dst = edge_index[1].reshape(nw, epw)
    if pad:
        src = jnp.concatenate(
            [src, jnp.zeros((nw, pad), jnp.int32)], axis=1)
        dst = jnp.concatenate(
            [dst, jnp.full((nw, pad), n, jnp.int32)], axis=1)
    src = src.reshape(nw, n_chunks, chunk)
    dst = dst.reshape(nw, n_chunks, chunk)
    z2d = jnp.zeros((n_pad, d_out), jnp.float32)
    z1d = jnp.zeros((n_pad,), jnp.float32)

    seg_sum = _make_seg_sum(n, n_pad, d_out, nw, n_chunks, chunk)
    agg, cnt = seg_sum(transformed, src, dst, z2d, z1d)

    return _finalize(agg, cnt[:, :n].T, block_n=1000)


# speedup vs baseline: 2.1472x; 2.1472x over previous
import functools

import jax
import jax.numpy as jnp
from jax import lax
from jax.experimental import pallas as pl
from jax.experimental.pallas import tpu as pltpu
from jax.experimental.pallas import tpu_sc as plsc


def _transform_body(x_ref, w_ref, b_ref, o_ref):
    x = x_ref[...]
    nrm = jnp.sqrt(jnp.sum(x * x, axis=1, keepdims=True))
    y = jnp.minimum(nrm, 1.0 - 1e-7)
    artanh = 0.5 * jnp.log((1.0 + y) / (1.0 - y))
    t = x * (artanh / jnp.maximum(nrm, 1e-15))
    o_ref[...] = (
        lax.dot_general(t, w_ref[...], (((1,), (1,)), ((), ())),
                        preferred_element_type=jnp.float32)
        + b_ref[...]
    )


def _transform(x, w, b, block_n):
    n, d_in = x.shape
    d_out = w.shape[0]
    return pl.pallas_call(
        _transform_body,
        grid=(n // block_n,),
        in_specs=[
            pl.BlockSpec((block_n, d_in), lambda i: (i, 0)),
            pl.BlockSpec((d_out, d_in), lambda i: (0, 0)),
            pl.BlockSpec((1, d_out), lambda i: (0, 0)),
        ],
        out_specs=pl.BlockSpec((block_n, d_out), lambda i: (i, 0)),
        out_shape=jax.ShapeDtypeStruct((n, d_out), jnp.float32),
    )(x, w, b.reshape(1, d_out))


def _make_seg_sum(n, n_pad, d, nw, n_chunks, chunk):
    mesh = plsc.VectorSubcoreMesh(core_axis_name="c", subcore_axis_name="s")
    ns = 16
    npt = (n_pad // ns) // 8 * 8
    remz = n_pad - ns * npt
    remo = n - ns * npt

    @functools.partial(
        pl.kernel,
        mesh=mesh,
        out_type=(
            jax.ShapeDtypeStruct((2, n, d), jnp.float32),
            jax.ShapeDtypeStruct((2, n_pad), jnp.float32),
        ),
        scratch_types=[
            pltpu.VMEM_SHARED((n_pad, d), jnp.float32),
            pltpu.VMEM_SHARED((n_pad,), jnp.float32),
            pltpu.VMEM((n_chunks, chunk), jnp.int32),
            pltpu.VMEM((n_chunks, chunk), jnp.int32),
            pltpu.VMEM((chunk, d), jnp.float32),
            pltpu.VMEM((128,), jnp.float32),
            pltpu.SemaphoreType.DMA,
            pltpu.SemaphoreType.DMA,
        ],
    )
    def seg_sum(t_hbm, src_hbm, dst_hbm, z2d_hbm, z1d_hbm,
                agg_hbm, cnt_hbm,
                acc_sh, cnt_sh, sidx_v, didx_v, rows_v, ones_v, gsem, csem):
        cc = lax.axis_index("c")
        s = lax.axis_index("s")
        wid = cc * ns + s

        pltpu.sync_copy(src_hbm.at[wid], sidx_v)
        pltpu.sync_copy(dst_hbm.at[wid], didx_v)

        pltpu.sync_copy(z2d_hbm.at[pl.ds(s * npt, npt)],
                        acc_sh.at[pl.ds(s * npt, npt)])

        @pl.when(s == 0)
        def _():
            pltpu.sync_copy(z1d_hbm, cnt_sh)
            if remz:
                pltpu.sync_copy(z2d_hbm.at[pl.ds(ns * npt, remz)],
                                acc_sh.at[pl.ds(ns * npt, remz)])

        for i in range(8):
            ones_v[pl.ds(i * 16, 16)] = jnp.full((16,), 1.0, jnp.float32)

        plsc.subcore_barrier()

        def body(j, carry):
            pltpu.async_copy(t_hbm.at[sidx_v.at[j]], rows_v, gsem).wait()
            pltpu.sync_copy(rows_v, acc_sh.at[didx_v.at[j]], add=True)
            pltpu.async_copy(ones_v.at[pl.ds(0, chunk)],
                             cnt_sh.at[didx_v.at[j]], csem, add=True)
            return carry

        lax.fori_loop(0, n_chunks, body, 0)

        def drain(j, carry):
            pltpu.make_async_copy(ones_v.at[pl.ds(0, chunk)],
                                  cnt_sh.at[didx_v.at[0]], csem).wait()
            return carry

        lax.fori_loop(0, n_chunks, drain, 0)

        plsc.subcore_barrier()

        pltpu.sync_copy(acc_sh.at[pl.ds(s * npt, npt)],
                        agg_hbm.at[cc, pl.ds(s * npt, npt)])

        @pl.when(s == 0)
        def _():
            pltpu.sync_copy(cnt_sh, cnt_hbm.at[cc])
            if remo:
                pltpu.sync_copy(acc_sh.at[pl.ds(ns * npt, remo)],
                                agg_hbm.at[cc, pl.ds(ns * npt, remo)])

    return seg_sum


def _finalize_body(agg_ref, cnt_ref, o_ref):
    a = agg_ref[0] + agg_ref[1]
    c = cnt_ref[...]
    csum = c[:, 0:1] + c[:, 1:2]
    neigh = a / jnp.maximum(csum, 1.0)
    nrm = jnp.sqrt(jnp.sum(neigh * neigh, axis=1, keepdims=True))
    o_ref[...] = jnp.tanh(nrm) * neigh / jnp.maximum(nrm, 1e-15)


def _finalize(agg, cnt_t, block_n):
    _, n, d = agg.shape
    return pl.pallas_call(
        _finalize_body,
        grid=(n // block_n,),
        in_specs=[
            pl.BlockSpec((2, block_n, d), lambda i: (0, i, 0)),
            pl.BlockSpec((block_n, 2), lambda i: (i, 0)),
        ],
        out_specs=pl.BlockSpec((block_n, d), lambda i: (i, 0)),
        out_shape=jax.ShapeDtypeStruct((n, d), jnp.float32),
    )(agg, cnt_t)


def kernel(x, edge_index, W, b):
    n, d_in = x.shape
    d_out = W.shape[0]
    e = edge_index.shape[1]

    nw = 32
    chunk = 125
    n_pad = n + 8
    epw = e // nw
    n_chunks = epw // chunk
    assert epw * nw == e and n_chunks * chunk == epw

    transformed = _transform(x, W, b, block_n=1000)

    src = edge_index[0].reshape(nw, n_chunks, chunk)
    dst = edge_index[1].reshape(nw, n_chunks, chunk)
    z2d = jnp.zeros((n_pad, d_out), jnp.float32)
    z1d = jnp.zeros((n_pad,), jnp.float32)

    seg_sum = _make_seg_sum(n, n_pad, d_out, nw, n_chunks, chunk)
    agg, cnt = seg_sum(transformed, src, dst, z2d, z1d)

    return _finalize(agg, cnt[:, :n].T, block_n=1000)


# speedup vs baseline: 2.9851x; 1.3902x over previous
import functools

import jax
import jax.numpy as jnp
from jax import lax
from jax.experimental import pallas as pl
from jax.experimental.pallas import tpu as pltpu
from jax.experimental.pallas import tpu_sc as plsc


def _transform_body(x_ref, w_ref, b_ref, o_ref):
    x = x_ref[...]
    nrm = jnp.sqrt(jnp.sum(x * x, axis=1, keepdims=True))
    y = jnp.minimum(nrm, 1.0 - 1e-7)
    artanh = 0.5 * jnp.log((1.0 + y) / (1.0 - y))
    t = x * (artanh / jnp.maximum(nrm, 1e-15))
    o_ref[...] = (
        lax.dot_general(t, w_ref[...], (((1,), (1,)), ((), ())),
                        preferred_element_type=jnp.float32)
        + b_ref[...]
    )


def _transform(x, w, b, block_n):
    n, d_in = x.shape
    d_out = w.shape[0]
    return pl.pallas_call(
        _transform_body,
        grid=(n // block_n,),
        in_specs=[
            pl.BlockSpec((block_n, d_in), lambda i: (i, 0)),
            pl.BlockSpec((d_out, d_in), lambda i: (0, 0)),
            pl.BlockSpec((1, d_out), lambda i: (0, 0)),
        ],
        out_specs=pl.BlockSpec((block_n, d_out), lambda i: (i, 0)),
        out_shape=jax.ShapeDtypeStruct((n, d_out), jnp.float32),
    )(x, w, b.reshape(1, d_out))


def _make_seg_sum(n, n_pad, d, nw, n_chunks, chunk):
    mesh = plsc.VectorSubcoreMesh(core_axis_name="c", subcore_axis_name="s")
    ns = 16
    npt = (n_pad // ns) // 8 * 8
    remz = n_pad - ns * npt
    remo = n - ns * npt

    nbuf = 2
    nring = 4
    assert n_chunks % nring == 0 and nring == 2 * nbuf

    @functools.partial(
        pl.kernel,
        mesh=mesh,
        out_type=(
            jax.ShapeDtypeStruct((2, n, d), jnp.float32),
            jax.ShapeDtypeStruct((2, n_pad), jnp.float32),
        ),
        scratch_types=[
            pltpu.VMEM_SHARED((n_pad, d), jnp.float32),
            pltpu.VMEM_SHARED((n_pad,), jnp.float32),
            pltpu.VMEM((nring, chunk), jnp.int32),
            pltpu.VMEM((n_chunks, chunk), jnp.int32),
        ]
        + [pltpu.VMEM((chunk, d), jnp.float32) for _ in range(nbuf)]
        + [pltpu.VMEM((128,), jnp.float32)]
        + [pltpu.SemaphoreType.DMA for _ in range(nbuf + 1 + nring)],
    )
    def seg_sum(t_hbm, src_hbm, dst_hbm, z2d_hbm, z1d_hbm,
                agg_hbm, cnt_hbm,
                acc_sh, cnt_sh, sring_v, didx_v, *bufs_ones_sems):
        rows = bufs_ones_sems[:nbuf]
        ones_v = bufs_ones_sems[nbuf]
        gsems = bufs_ones_sems[nbuf + 1:nbuf + 1 + nbuf]
        csem = bufs_ones_sems[nbuf + 1 + nbuf]
        isems = bufs_ones_sems[nbuf + 2 + nbuf:]
        cc = lax.axis_index("c")
        s = lax.axis_index("s")
        wid = cc * ns + s

        pltpu.sync_copy(dst_hbm.at[wid], didx_v)

        pltpu.sync_copy(z2d_hbm.at[pl.ds(s * npt, npt)],
                        acc_sh.at[pl.ds(s * npt, npt)])

        @pl.when(s == 0)
        def _():
            pltpu.sync_copy(z1d_hbm, cnt_sh)
            if remz:
                pltpu.sync_copy(z2d_hbm.at[pl.ds(ns * npt, remz)],
                                acc_sh.at[pl.ds(ns * npt, remz)])

        for i in range(8):
            ones_v[pl.ds(i * 16, 16)] = jnp.full((16,), 1.0, jnp.float32)

        plsc.subcore_barrier()

        for k in range(nring):
            pltpu.async_copy(src_hbm.at[wid, k, pl.ds(0, chunk)],
                             sring_v.at[k], isems[k])
        for b in range(nbuf):
            pltpu.make_async_copy(src_hbm.at[wid, b, pl.ds(0, chunk)],
                                  sring_v.at[b], isems[b]).wait()
            pltpu.async_copy(t_hbm.at[sring_v.at[b]], rows[b], gsems[b])

        def body(it, carry):
            for u in range(nring):
                jj = it * nring + u
                b = u % nbuf

                pltpu.make_async_copy(t_hbm.at[sring_v.at[u]],
                                      rows[b], gsems[b]).wait()
                pltpu.sync_copy(rows[b], acc_sh.at[didx_v.at[jj]], add=True)
                pltpu.async_copy(ones_v.at[pl.ds(0, chunk)],
                                 cnt_sh.at[didx_v.at[jj]], csem, add=True)

                @pl.when(jj + nring < n_chunks)
                def _():
                    pltpu.async_copy(src_hbm.at[wid, jj + nring,
                                                pl.ds(0, chunk)],
                                     sring_v.at[u], isems[u])

                @pl.when(jj + nbuf < n_chunks)
                def _():
                    u2 = (u + nbuf) % nring
                    pltpu.make_async_copy(src_hbm.at[wid, 0, pl.ds(0, chunk)],
                                          sring_v.at[u2], isems[u2]).wait()
                    pltpu.async_copy(t_hbm.at[sring_v.at[u2]],
                                     rows[b], gsems[b])

            return carry

        lax.fori_loop(0, n_chunks // nring, body, 0)

        def drain(j, carry):
            pltpu.make_async_copy(ones_v.at[pl.ds(0, chunk)],
                                  cnt_sh.at[didx_v.at[0]], csem).wait()
            return carry

        lax.fori_loop(0, n_chunks, drain, 0)

        plsc.subcore_barrier()

        pltpu.sync_copy(acc_sh.at[pl.ds(s * npt, npt)],
                        agg_hbm.at[cc, pl.ds(s * npt, npt)])

        @pl.when(s == 0)
        def _():
            pltpu.sync_copy(cnt_sh, cnt_hbm.at[cc])
            if remo:
                pltpu.sync_copy(acc_sh.at[pl.ds(ns * npt, remo)],
                                agg_hbm.at[cc, pl.ds(ns * npt, remo)])

    return seg_sum


def _finalize_body(agg_ref, cnt_ref, o_ref):
    a = agg_ref[0] + agg_ref[1]
    c = cnt_ref[...]
    csum = c[:, 0:1] + c[:, 1:2]
    neigh = a / jnp.maximum(csum, 1.0)
    nrm = jnp.sqrt(jnp.sum(neigh * neigh, axis=1, keepdims=True))
    o_ref[...] = jnp.tanh(nrm) * neigh / jnp.maximum(nrm, 1e-15)


def _finalize(agg, cnt_t, block_n):
    _, n, d = agg.shape
    return pl.pallas_call(
        _finalize_body,
        grid=(n // block_n,),
        in_specs=[
            pl.BlockSpec((2, block_n, d), lambda i: (0, i, 0)),
            pl.BlockSpec((block_n, 2), lambda i: (i, 0)),
        ],
        out_specs=pl.BlockSpec((block_n, d), lambda i: (i, 0)),
        out_shape=jax.ShapeDtypeStruct((n, d), jnp.float32),
    )(agg, cnt_t)


def kernel(x, edge_index, W, b):
    n, d_in = x.shape
    d_out = W.shape[0]
    e = edge_index.shape[1]

    nw = 32
    chunk = 125
    n_pad = n + 8
    epw = e // nw
    n_chunks = epw // chunk
    assert epw * nw == e and n_chunks * chunk == epw

    transformed = _transform(x, W, b, block_n=1000)

    src = jnp.pad(edge_index[0].reshape(nw, n_chunks, chunk),
                  ((0, 0), (0, 0), (0, 128 - chunk)))
    dst = edge_index[1].reshape(nw, n_chunks, chunk)
    z2d = jnp.zeros((n_pad, d_out), jnp.float32)
    z1d = jnp.zeros((n_pad,), jnp.float32)

    seg_sum = _make_seg_sum(n, n_pad, d_out, nw, n_chunks, chunk)
    agg, cnt = seg_sum(transformed, src, dst, z2d, z1d)

    return _finalize(agg, cnt[:, :n].T, block_n=1000)


# speedup vs baseline: 2.9884x; 1.0011x over previous
import functools

import jax
import jax.numpy as jnp
from jax import lax
from jax.experimental import pallas as pl
from jax.experimental.pallas import tpu as pltpu
from jax.experimental.pallas import tpu_sc as plsc


def _transform_body(x_ref, w_ref, b_ref, o_ref):
    x = x_ref[...]
    nrm = jnp.sqrt(jnp.sum(x * x, axis=1, keepdims=True))
    y = jnp.minimum(nrm, 1.0 - 1e-7)
    artanh = 0.5 * jnp.log((1.0 + y) / (1.0 - y))
    t = x * (artanh / jnp.maximum(nrm, 1e-15))
    o_ref[...] = (
        lax.dot_general(t, w_ref[...], (((1,), (1,)), ((), ())),
                        preferred_element_type=jnp.float32)
        + b_ref[...]
    )


def _transform(x, w, b, block_n):
    n, d_in = x.shape
    d_out = w.shape[0]
    return pl.pallas_call(
        _transform_body,
        grid=(n // block_n,),
        in_specs=[
            pl.BlockSpec((block_n, d_in), lambda i: (i, 0)),
            pl.BlockSpec((d_out, d_in), lambda i: (0, 0)),
            pl.BlockSpec((1, d_out), lambda i: (0, 0)),
        ],
        out_specs=pl.BlockSpec((block_n, d_out), lambda i: (i, 0)),
        out_shape=jax.ShapeDtypeStruct((n, d_out), jnp.float32),
    )(x, w, b.reshape(1, d_out))


def _make_seg_sum(n, n_pad, d, nw, n_chunks, chunk):
    mesh = plsc.VectorSubcoreMesh(core_axis_name="c", subcore_axis_name="s")
    ns = 16
    npt = (n_pad // ns) // 8 * 8
    remz = n_pad - ns * npt
    remo = n - ns * npt

    nbuf = 2
    nring = 4
    assert n_chunks % nring == 0 and nring == 2 * nbuf

    @functools.partial(
        pl.kernel,
        mesh=mesh,
        out_type=(
            jax.ShapeDtypeStruct((2, n, d), jnp.float32),
            jax.ShapeDtypeStruct((2, n_pad), jnp.float32),
        ),
        scratch_types=[
            pltpu.VMEM_SHARED((n_pad, d), jnp.float32),
            pltpu.VMEM_SHARED((n_pad,), jnp.float32),
            pltpu.VMEM((nring, chunk), jnp.int32),
            pltpu.VMEM((n_chunks, chunk), jnp.int32),
        ]
        + [pltpu.VMEM((chunk, d), jnp.float32) for _ in range(nbuf)]
        + [pltpu.VMEM((128,), jnp.float32)]
        + [pltpu.SemaphoreType.DMA for _ in range(nbuf + 1 + nring)],
    )
    def seg_sum(t_hbm, src_hbm, dst_hbm, z2d_hbm, z1d_hbm,
                agg_hbm, cnt_hbm,
                acc_sh, cnt_sh, sring_v, didx_v, *bufs_ones_sems):
        rows = bufs_ones_sems[:nbuf]
        ones_v = bufs_ones_sems[nbuf]
        gsems = bufs_ones_sems[nbuf + 1:nbuf + 1 + nbuf]
        csem = bufs_ones_sems[nbuf + 1 + nbuf]
        isems = bufs_ones_sems[nbuf + 2 + nbuf:]
        cc = lax.axis_index("c")
        s = lax.axis_index("s")
        wid = cc * ns + s

        pltpu.sync_copy(dst_hbm.at[wid], didx_v)

        pltpu.sync_copy(z2d_hbm.at[pl.ds(s * npt, npt)],
                        acc_sh.at[pl.ds(s * npt, npt)])

        @pl.when(s == 0)
        def _():
            pltpu.sync_copy(z1d_hbm, cnt_sh)
            if remz:
                pltpu.sync_copy(z2d_hbm.at[pl.ds(ns * npt, remz)],
                                acc_sh.at[pl.ds(ns * npt, remz)])

        for i in range(8):
            ones_v[pl.ds(i * 16, 16)] = jnp.full((16,), 1.0, jnp.float32)

        plsc.subcore_barrier()

        for k in range(nring):
            pltpu.async_copy(src_hbm.at[wid, k, pl.ds(0, chunk)],
                             sring_v.at[k], isems[k])
        for b in range(nbuf):
            pltpu.make_async_copy(src_hbm.at[wid, b, pl.ds(0, chunk)],
                                  sring_v.at[b], isems[b]).wait()
            pltpu.async_copy(t_hbm.at[sring_v.at[b]], rows[b], gsems[b])

        def body(it, carry):
            for u in range(nring):
                jj = it * nring + u
                b = u % nbuf

                pltpu.make_async_copy(t_hbm.at[sring_v.at[u]],
                                      rows[b], gsems[b]).wait()
                pltpu.sync_copy(rows[b], acc_sh.at[didx_v.at[jj]], add=True)
                pltpu.async_copy(ones_v.at[pl.ds(0, chunk)],
                                 cnt_sh.at[didx_v.at[jj]], csem, add=True)

                @pl.when(jj + nring < n_chunks)
                def _():
                    pltpu.async_copy(src_hbm.at[wid, jj + nring,
                                                pl.ds(0, chunk)],
                                     sring_v.at[u], isems[u])

                @pl.when(jj + nbuf < n_chunks)
                def _():
                    u2 = (u + nbuf) % nring
                    pltpu.make_async_copy(src_hbm.at[wid, 0, pl.ds(0, chunk)],
                                          sring_v.at[u2], isems[u2]).wait()
                    pltpu.async_copy(t_hbm.at[sring_v.at[u2]],
                                     rows[b], gsems[b])

            return carry

        lax.fori_loop(0, n_chunks // nring, body, 0)

        def drain(j, carry):
            pltpu.make_async_copy(ones_v.at[pl.ds(0, chunk)],
                                  cnt_sh.at[didx_v.at[0]], csem).wait()
            return carry

        lax.fori_loop(0, n_chunks, drain, 0)

        plsc.subcore_barrier()

        pltpu.sync_copy(acc_sh.at[pl.ds(s * npt, npt)],
                        agg_hbm.at[cc, pl.ds(s * npt, npt)])

        @pl.when(s == 0)
        def _():
            pltpu.sync_copy(cnt_sh, cnt_hbm.at[cc])
            if remo:
                pltpu.sync_copy(acc_sh.at[pl.ds(ns * npt, remo)],
                                agg_hbm.at[cc, pl.ds(ns * npt, remo)])

    return seg_sum


def _finalize_body(block_n, agg_ref, cnt_ref, o_ref):
    a = agg_ref[0] + agg_ref[1]
    c = cnt_ref[...]
    csum = jnp.sum(c, axis=1, keepdims=True)
    neigh = a / jnp.maximum(csum, 1.0)
    nrm = jnp.sqrt(jnp.sum(neigh * neigh, axis=1, keepdims=True))
    o_ref[...] = jnp.tanh(nrm) * neigh / jnp.maximum(nrm, 1e-15)


def _finalize(agg, cnt, block_n):
    _, n, d = agg.shape
    n_pad = cnt.shape[1]
    return pl.pallas_call(
        functools.partial(_finalize_body, block_n),
        grid=(n // block_n,),
        in_specs=[
            pl.BlockSpec((2, block_n, d), lambda i: (0, i, 0)),
            pl.BlockSpec((block_n, 2), lambda i: (i, 0)),
        ],
        out_specs=pl.BlockSpec((block_n, d), lambda i: (i, 0)),
        out_shape=jax.ShapeDtypeStruct((n, d), jnp.float32),
    )(agg, cnt)


def kernel(x, edge_index, W, b):
    n, d_in = x.shape
    d_out = W.shape[0]
    e = edge_index.shape[1]

    nw = 32
    chunk = 125
    n_pad = n
    epw = e // nw
    n_chunks = epw // chunk
    assert epw * nw == e and n_chunks * chunk == epw

    transformed = _transform(x, W, b, block_n=1000)

    src = jnp.pad(edge_index[0].reshape(nw, n_chunks, chunk),
                  ((0, 0), (0, 0), (0, 128 - chunk)))
    dst = edge_index[1].reshape(nw, n_chunks, chunk)
    z2d = jnp.zeros((n_pad, d_out), jnp.float32)
    z1d = jnp.zeros((n_pad,), jnp.float32)

    seg_sum = _make_seg_sum(n, n_pad, d_out, nw, n_chunks, chunk)
    agg, cnt = seg_sum(transformed, src, dst, z2d, z1d)

    return _finalize(agg, cnt.T, block_n=1000)


# speedup vs baseline: 3.0010x; 1.0042x over previous
import functools

import jax
import jax.numpy as jnp
from jax import lax
from jax.experimental import pallas as pl
from jax.experimental.pallas import tpu as pltpu
from jax.experimental.pallas import tpu_sc as plsc


def _transform_body(x_ref, w_ref, b_ref, o_ref):
    x = x_ref[...]
    nrm = jnp.sqrt(jnp.sum(x * x, axis=1, keepdims=True))
    y = jnp.minimum(nrm, 1.0 - 1e-7)
    artanh = 0.5 * jnp.log((1.0 + y) / (1.0 - y))
    t = x * (artanh / jnp.maximum(nrm, 1e-15))
    o_ref[...] = (
        lax.dot_general(t, w_ref[...], (((1,), (1,)), ((), ())),
                        preferred_element_type=jnp.float32)
        + b_ref[...]
    )


def _transform(x, w, b, block_n):
    n, d_in = x.shape
    d_out = w.shape[0]
    return pl.pallas_call(
        _transform_body,
        grid=(n // block_n,),
        in_specs=[
            pl.BlockSpec((block_n, d_in), lambda i: (i, 0)),
            pl.BlockSpec((d_out, d_in), lambda i: (0, 0)),
            pl.BlockSpec((1, d_out), lambda i: (0, 0)),
        ],
        out_specs=pl.BlockSpec((block_n, d_out), lambda i: (i, 0)),
        out_shape=jax.ShapeDtypeStruct((n, d_out), jnp.float32),
    )(x, w, b.reshape(1, d_out))


def _make_seg_sum(n, n_pad, d, nw, n_chunks, chunk):
    mesh = plsc.VectorSubcoreMesh(core_axis_name="c", subcore_axis_name="s")
    ns = 16
    npt = (n_pad // ns) // 8 * 8
    remz = n_pad - ns * npt
    remo = n - ns * npt

    nbuf = 2
    nring = 4
    assert n_chunks % nring == 0 and nring == 2 * nbuf

    @functools.partial(
        pl.kernel,
        mesh=mesh,
        out_type=(
            jax.ShapeDtypeStruct((2, n, d), jnp.float32),
            jax.ShapeDtypeStruct((2, n_pad), jnp.float32),
        ),
        scratch_types=[
            pltpu.VMEM_SHARED((n_pad, d), jnp.float32),
            pltpu.VMEM_SHARED((n_pad,), jnp.float32),
            pltpu.VMEM((nring, chunk), jnp.int32),
            pltpu.VMEM((n_chunks, chunk), jnp.int32),
        ]
        + [pltpu.VMEM((chunk, d), jnp.float32) for _ in range(nbuf)]
        + [pltpu.VMEM((128,), jnp.float32)]
        + [pltpu.SemaphoreType.DMA for _ in range(nbuf + 1 + nring)],
    )
    def seg_sum(t_hbm, src_hbm, dst_hbm, z2d_hbm, z1d_hbm,
                agg_hbm, cnt_hbm,
                acc_sh, cnt_sh, sring_v, didx_v, *bufs_ones_sems):
        rows = bufs_ones_sems[:nbuf]
        ones_v = bufs_ones_sems[nbuf]
        gsems = bufs_ones_sems[nbuf + 1:nbuf + 1 + nbuf]
        csem = bufs_ones_sems[nbuf + 1 + nbuf]
        isems = bufs_ones_sems[nbuf + 2 + nbuf:]
        cc = lax.axis_index("c")
        s = lax.axis_index("s")
        wid = cc * ns + s

        pltpu.sync_copy(dst_hbm.at[wid], didx_v)

        pltpu.sync_copy(z2d_hbm.at[pl.ds(s * npt, npt)],
                        acc_sh.at[pl.ds(s * npt, npt)])

        @pl.when(s == 0)
        def _():
            pltpu.sync_copy(z1d_hbm, cnt_sh)
            if remz:
                pltpu.sync_copy(z2d_hbm.at[pl.ds(ns * npt, remz)],
                                acc_sh.at[pl.ds(ns * npt, remz)])

        for i in range(8):
            ones_v[pl.ds(i * 16, 16)] = jnp.full((16,), 1.0, jnp.float32)

        plsc.subcore_barrier()

        for k in range(nring):
            pltpu.async_copy(src_hbm.at[wid, k, pl.ds(0, chunk)],
                             sring_v.at[k], isems[k])
        for b in range(nbuf):
            pltpu.make_async_copy(src_hbm.at[wid, b, pl.ds(0, chunk)],
                                  sring_v.at[b], isems[b]).wait()
            pltpu.async_copy(t_hbm.at[sring_v.at[b]], rows[b], gsems[b])

        def body(it, carry):
            for u in range(nring):
                jj = it * nring + u
                b = u % nbuf

                pltpu.make_async_copy(t_hbm.at[sring_v.at[u]],
                                      rows[b], gsems[b]).wait()
                pltpu.sync_copy(rows[b], acc_sh.at[didx_v.at[jj]], add=True)

                @pl.when(jj + nbuf < n_chunks)
                def _():
                    u2 = (u + nbuf) % nring
                    pltpu.make_async_copy(src_hbm.at[wid, 0, pl.ds(0, chunk)],
                                          sring_v.at[u2], isems[u2]).wait()
                    pltpu.async_copy(t_hbm.at[sring_v.at[u2]],
                                     rows[b], gsems[b])

                pltpu.async_copy(ones_v.at[pl.ds(0, chunk)],
                                 cnt_sh.at[didx_v.at[jj]], csem, add=True)

                @pl.when(jj + nring < n_chunks)
                def _():
                    pltpu.async_copy(src_hbm.at[wid, jj + nring,
                                                pl.ds(0, chunk)],
                                     sring_v.at[u], isems[u])

            return carry

        lax.fori_loop(0, n_chunks // nring, body, 0)

        def drain(j, carry):
            pltpu.make_async_copy(ones_v.at[pl.ds(0, chunk)],
                                  cnt_sh.at[didx_v.at[0]], csem).wait()
            return carry

        lax.fori_loop(0, n_chunks, drain, 0)

        plsc.subcore_barrier()

        pltpu.sync_copy(acc_sh.at[pl.ds(s * npt, npt)],
                        agg_hbm.at[cc, pl.ds(s * npt, npt)])

        @pl.when(s == 0)
        def _():
            pltpu.sync_copy(cnt_sh, cnt_hbm.at[cc])
            if remo:
                pltpu.sync_copy(acc_sh.at[pl.ds(ns * npt, remo)],
                                agg_hbm.at[cc, pl.ds(ns * npt, remo)])

    return seg_sum


def _finalize_body(block_n, agg_ref, cnt_ref, o_ref):
    a = agg_ref[0] + agg_ref[1]
    c = cnt_ref[...]
    csum = jnp.sum(c, axis=1, keepdims=True)
    neigh = a / jnp.maximum(csum, 1.0)
    nrm = jnp.sqrt(jnp.sum(neigh * neigh, axis=1, keepdims=True))
    o_ref[...] = jnp.tanh(nrm) * neigh / jnp.maximum(nrm, 1e-15)


def _finalize(agg, cnt, block_n):
    _, n, d = agg.shape
    n_pad = cnt.shape[1]
    return pl.pallas_call(
        functools.partial(_finalize_body, block_n),
        grid=(n // block_n,),
        in_specs=[
            pl.BlockSpec((2, block_n, d), lambda i: (0, i, 0)),
            pl.BlockSpec((block_n, 2), lambda i: (i, 0)),
        ],
        out_specs=pl.BlockSpec((block_n, d), lambda i: (i, 0)),
        out_shape=jax.ShapeDtypeStruct((n, d), jnp.float32),
    )(agg, cnt)


def kernel(x, edge_index, W, b):
    n, d_in = x.shape
    d_out = W.shape[0]
    e = edge_index.shape[1]

    nw = 32
    chunk = 125
    n_pad = n
    epw = e // nw
    n_chunks = epw // chunk
    assert epw * nw == e and n_chunks * chunk == epw

    transformed = _transform(x, W, b, block_n=1000)

    src = jnp.pad(edge_index[0].reshape(nw, n_chunks, chunk),
                  ((0, 0), (0, 0), (0, 128 - chunk)))
    dst = edge_index[1].reshape(nw, n_chunks, chunk)
    z2d = jnp.zeros((n_pad, d_out), jnp.float32)
    z1d = jnp.zeros((n_pad,), jnp.float32)

    seg_sum = _make_seg_sum(n, n_pad, d_out, nw, n_chunks, chunk)
    agg, cnt = seg_sum(transformed, src, dst, z2d, z1d)

    return _finalize(agg, cnt.T, block_n=1000)


# speedup vs baseline: 3.0035x; 1.0008x over previous
import functools

import jax
import jax.numpy as jnp
from jax import lax
from jax.experimental import pallas as pl
from jax.experimental.pallas import tpu as pltpu
from jax.experimental.pallas import tpu_sc as plsc


def _transform_body(x_ref, w_ref, b_ref, o_ref):
    x = x_ref[...]
    nrm = jnp.sqrt(jnp.sum(x * x, axis=1, keepdims=True))
    y = jnp.minimum(nrm, 1.0 - 1e-7)
    artanh = 0.5 * jnp.log((1.0 + y) / (1.0 - y))
    t = x * (artanh / jnp.maximum(nrm, 1e-15))
    o_ref[...] = (
        lax.dot_general(t, w_ref[...], (((1,), (1,)), ((), ())),
                        preferred_element_type=jnp.float32)
        + b_ref[...]
    )


def _transform(x, w, b, block_n):
    n, d_in = x.shape
    d_out = w.shape[0]
    return pl.pallas_call(
        _transform_body,
        grid=(n // block_n,),
        in_specs=[
            pl.BlockSpec((block_n, d_in), lambda i: (i, 0)),
            pl.BlockSpec((d_out, d_in), lambda i: (0, 0)),
            pl.BlockSpec((1, d_out), lambda i: (0, 0)),
        ],
        out_specs=pl.BlockSpec((block_n, d_out), lambda i: (i, 0)),
        out_shape=jax.ShapeDtypeStruct((n, d_out), jnp.float32),
    )(x, w, b.reshape(1, d_out))


def _make_seg_sum(n, n_pad, d, nw, n_chunks, chunk):
    mesh = plsc.VectorSubcoreMesh(core_axis_name="c", subcore_axis_name="s")
    ns = 16
    npt = (n_pad // ns) // 8 * 8
    remz = n_pad - ns * npt
    remo = n - ns * npt

    nbuf = 2
    nring = 4
    assert n_chunks % nring == 0 and nring == 2 * nbuf

    @functools.partial(
        pl.kernel,
        mesh=mesh,
        out_type=(
            jax.ShapeDtypeStruct((2, n, d), jnp.float32),
            jax.ShapeDtypeStruct((2, n_pad), jnp.float32),
        ),
        scratch_types=[
            pltpu.VMEM_SHARED((n_pad, d), jnp.float32),
            pltpu.VMEM_SHARED((n_pad,), jnp.float32),
            pltpu.VMEM((nring, chunk), jnp.int32),
            pltpu.VMEM((n_chunks, chunk), jnp.int32),
        ]
        + [pltpu.VMEM((chunk, d), jnp.float32) for _ in range(nbuf)]
        + [pltpu.VMEM((128,), jnp.float32)]
        + [pltpu.SemaphoreType.DMA for _ in range(nbuf + 1 + nring)],
    )
    def seg_sum(t_hbm, src_hbm, dst_hbm, z2d_hbm, z1d_hbm,
                agg_hbm, cnt_hbm,
                acc_sh, cnt_sh, sring_v, didx_v, *bufs_ones_sems):
        rows = bufs_ones_sems[:nbuf]
        ones_v = bufs_ones_sems[nbuf]
        gsems = bufs_ones_sems[nbuf + 1:nbuf + 1 + nbuf]
        csem = bufs_ones_sems[nbuf + 1 + nbuf]
        isems = bufs_ones_sems[nbuf + 2 + nbuf:]
        cc = lax.axis_index("c")
        s = lax.axis_index("s")
        wid = cc * ns + s

        pltpu.sync_copy(dst_hbm.at[wid], didx_v)

        pltpu.sync_copy(z2d_hbm.at[pl.ds(s * npt, npt)],
                        acc_sh.at[pl.ds(s * npt, npt)])

        @pl.when(s == 0)
        def _():
            pltpu.sync_copy(z1d_hbm, cnt_sh)
            if remz:
                pltpu.sync_copy(z2d_hbm.at[pl.ds(ns * npt, remz)],
                                acc_sh.at[pl.ds(ns * npt, remz)])

        for i in range(8):
            ones_v[pl.ds(i * 16, 16)] = jnp.full((16,), 1.0, jnp.float32)

        plsc.subcore_barrier()

        for k in range(nring):
            pltpu.async_copy(src_hbm.at[wid, k, pl.ds(0, chunk)],
                             sring_v.at[k], isems[k])
        for b in range(nbuf):
            pltpu.make_async_copy(src_hbm.at[wid, b, pl.ds(0, chunk)],
                                  sring_v.at[b], isems[b]).wait()
            pltpu.async_copy(t_hbm.at[sring_v.at[b]], rows[b], gsems[b])

        def body(it, carry):
            for u in range(nring):
                jj = it * nring + u
                b = u % nbuf

                pltpu.make_async_copy(t_hbm.at[sring_v.at[u]],
                                      rows[b], gsems[b]).wait()
                pltpu.sync_copy(rows[b], acc_sh.at[didx_v.at[jj]], add=True)

                @pl.when(jj + nbuf < n_chunks)
                def _():
                    u2 = (u + nbuf) % nring
                    pltpu.make_async_copy(src_hbm.at[wid, 0, pl.ds(0, chunk)],
                                          sring_v.at[u2], isems[u2]).wait()
                    pltpu.async_copy(t_hbm.at[sring_v.at[u2]],
                                     rows[b], gsems[b])

                pltpu.async_copy(ones_v.at[pl.ds(0, chunk)],
                                 cnt_sh.at[didx_v.at[jj]], csem, add=True)

                @pl.when(jj + nring < n_chunks)
                def _():
                    pltpu.async_copy(src_hbm.at[wid, jj + nring,
                                                pl.ds(0, chunk)],
                                     sring_v.at[u], isems[u])

            return carry

        lax.fori_loop(0, n_chunks // nring, body, 0)

        def drain(j, carry):
            pltpu.make_async_copy(ones_v.at[pl.ds(0, chunk)],
                                  cnt_sh.at[didx_v.at[0]], csem).wait()
            return carry

        lax.fori_loop(0, n_chunks, drain, 0)

        plsc.subcore_barrier()

        pltpu.sync_copy(acc_sh.at[pl.ds(s * npt, npt)],
                        agg_hbm.at[cc, pl.ds(s * npt, npt)])

        @pl.when(s == 0)
        def _():
            pltpu.sync_copy(cnt_sh, cnt_hbm.at[cc])
            if remo:
                pltpu.sync_copy(acc_sh.at[pl.ds(ns * npt, remo)],
                                agg_hbm.at[cc, pl.ds(ns * npt, remo)])

    return seg_sum


def _finalize_body(block_n, agg_ref, cnt_ref, o_ref):
    a = agg_ref[0] + agg_ref[1]
    c = cnt_ref[...]
    csum = jnp.sum(c, axis=1, keepdims=True)
    neigh = a / jnp.maximum(csum, 1.0)
    nrm = jnp.sqrt(jnp.sum(neigh * neigh, axis=1, keepdims=True))
    o_ref[...] = jnp.tanh(nrm) * neigh / jnp.maximum(nrm, 1e-15)


def _finalize(agg, cnt, block_n):
    _, n, d = agg.shape
    return pl.pallas_call(
        functools.partial(_finalize_body, block_n),
        grid=(n // block_n,),
        in_specs=[
            pl.BlockSpec((2, block_n, d), lambda i: (0, i, 0)),
            pl.BlockSpec((block_n, 2), lambda i: (i, 0)),
        ],
        out_specs=pl.BlockSpec((block_n, d), lambda i: (i, 0)),
        out_shape=jax.ShapeDtypeStruct((n, d), jnp.float32),
    )(agg, cnt)


def kernel(x, edge_index, W, b):
    n, d_in = x.shape
    d_out = W.shape[0]
    e = edge_index.shape[1]

    nw = 32
    chunk = 125
    n_pad = n
    epw = e // nw
    n_chunks = epw // chunk
    assert epw * nw == e and n_chunks * chunk == epw

    transformed = _transform(x, W, b, block_n=1000)

    src = jnp.pad(edge_index[0].reshape(nw, n_chunks, chunk),
                  ((0, 0), (0, 0), (0, 128 - chunk)))
    dst = edge_index[1].reshape(nw, n_chunks, chunk)
    z2d = jnp.zeros((n_pad, d_out), jnp.float32)
    z1d = jnp.zeros((n_pad,), jnp.float32)

    seg_sum = _make_seg_sum(n, n_pad, d_out, nw, n_chunks, chunk)
    agg, cnt = seg_sum(transformed, src, dst, z2d, z1d)

    return _finalize(agg, cnt.T, block_n=1000)
